# baseline (speedup 1.0000x reference)
import functools

import jax
import jax.numpy as jnp
from jax import lax
from jax.experimental import pallas as pl
from jax.experimental.pallas import tpu as pltpu
from jax.experimental.pallas import tpu_sc as plsc

DIM = 64
LANES = 16
NCORES = 2
NSUBCORES = 16
NW = NCORES * NSUBCORES
IDX_MINOR = 128


@functools.partial(jax.jit, static_argnames=("batch",))
def _sc_sumsq(ent_emb, rel_emb, idx_h, idx_r, idx_t, batch):
    b_per_w = batch // NW
    n_chunks = b_per_w // IDX_MINOR
    mesh = plsc.VectorSubcoreMesh(
        core_axis_name="c", subcore_axis_name="s",
        num_cores=NCORES, num_subcores=NSUBCORES,
    )

    @functools.partial(
        pl.kernel,
        out_type=jax.ShapeDtypeStruct((batch,), jnp.float32),
        mesh=mesh,
        compiler_params=pltpu.CompilerParams(
            use_tc_tiling_on_sc=False, needs_layout_passes=False),
        scratch_types=[
            pltpu.VMEM((n_chunks, IDX_MINOR), jnp.int32),
            pltpu.VMEM((n_chunks, IDX_MINOR), jnp.int32),
            pltpu.VMEM((n_chunks, IDX_MINOR), jnp.int32),
            pltpu.VMEM((b_per_w, DIM), jnp.float32),
            pltpu.VMEM((b_per_w, DIM), jnp.float32),
            pltpu.VMEM((b_per_w, DIM), jnp.float32),
            pltpu.VMEM((b_per_w, LANES), jnp.float32),
            pltpu.VMEM((b_per_w,), jnp.float32),
            pltpu.SemaphoreType.DMA,
        ],
    )
    def sc_kernel(ent_hbm, rel_hbm, ih_hbm, ir_hbm, it_hbm, out_hbm,
                  ih_v, ir_v, it_v, h_v, r_v, t_v, acc_v, sum_v, sem):
        wid = lax.axis_index("s") * NCORES + lax.axis_index("c")
        row0 = wid * n_chunks

        pltpu.sync_copy(ih_hbm.at[pl.ds(row0, n_chunks)], ih_v)
        pltpu.sync_copy(ir_hbm.at[pl.ds(row0, n_chunks)], ir_v)
        pltpu.sync_copy(it_hbm.at[pl.ds(row0, n_chunks)], it_v)

        copies = []
        for j in range(n_chunks):
            dst = pl.ds(j * IDX_MINOR, IDX_MINOR)
            copies.append(pltpu.async_copy(ent_hbm.at[ih_v.at[j]], h_v.at[dst], sem))
            copies.append(pltpu.async_copy(rel_hbm.at[ir_v.at[j]], r_v.at[dst], sem))
            copies.append(pltpu.async_copy(ent_hbm.at[it_v.at[j]], t_v.at[dst], sem))
        for c in copies:
            c.wait()

        @pl.loop(0, b_per_w)
        def _(i):
            acc = None
            for c in range(DIM // LANES):
                sl = pl.ds(c * LANES, LANES)
                d = h_v[i, sl] + r_v[i, sl] - t_v[i, sl]
                acc = d * d if acc is None else acc + d * d
            acc_v[i, :] = acc

        lane_iota = lax.iota(jnp.int32, LANES)

        @pl.loop(0, b_per_w // LANES)
        def _(g):
            rows = g * LANES + lane_iota
            tot = None
            for c in range(LANES):
                cols = jnp.full((LANES,), c, jnp.int32)
                v = plsc.load_gather(acc_v, [rows, cols])
                tot = v if tot is None else tot + v
            sum_v[pl.ds(g * LANES, LANES)] = tot

        pltpu.sync_copy(sum_v, out_hbm.at[pl.ds(wid * b_per_w, b_per_w)])

    return sc_kernel(ent_emb, rel_emb, idx_h, idx_r, idx_t)


def _tc_score_body(p_ref, o_ref):
    o_ref[...] = -jnp.sqrt(p_ref[...] + 1e-12)


def kernel(batch_h, batch_r, batch_t, ent_emb, rel_emb):
    batch = batch_h.shape[0]
    shp = (batch // IDX_MINOR, IDX_MINOR)
    ih = batch_h.astype(jnp.int32).reshape(shp)
    ir = batch_r.astype(jnp.int32).reshape(shp)
    it = batch_t.astype(jnp.int32).reshape(shp)
    sumsq = _sc_sumsq(ent_emb, rel_emb, ih, ir, it, batch)
    return pl.pallas_call(
        _tc_score_body,
        out_shape=jax.ShapeDtypeStruct((batch,), jnp.float32),
    )(sumsq)


# speedup vs baseline: 1.0003x; 1.0003x over previous
import functools

import jax
import jax.numpy as jnp
from jax import lax
from jax.experimental import pallas as pl
from jax.experimental.pallas import tpu as pltpu
from jax.experimental.pallas import tpu_sc as plsc

DIM = 64
LANES = 16
NCORES = 2
NSUBCORES = 16
NW = NCORES * NSUBCORES
IDX_MINOR = 128


@functools.partial(jax.jit, static_argnames=("batch",))
def _sc_sumsq(ent_emb, rel_emb, idx_h, idx_r, idx_t, batch):
    b_per_w = batch // NW
    n_chunks = b_per_w // IDX_MINOR
    mesh = plsc.VectorSubcoreMesh(
        core_axis_name="c", subcore_axis_name="s",
        num_cores=NCORES, num_subcores=NSUBCORES,
    )

    @functools.partial(
        pl.kernel,
        out_type=jax.ShapeDtypeStruct((batch,), jnp.float32),
        mesh=mesh,
        compiler_params=pltpu.CompilerParams(
            use_tc_tiling_on_sc=False, needs_layout_passes=False),
        scratch_types=[
            pltpu.VMEM((n_chunks, IDX_MINOR), jnp.int32),
            pltpu.VMEM((n_chunks, IDX_MINOR), jnp.int32),
            pltpu.VMEM((n_chunks, IDX_MINOR), jnp.int32),
            pltpu.VMEM((b_per_w, DIM), jnp.float32),
            pltpu.VMEM((b_per_w, DIM), jnp.float32),
            pltpu.VMEM((b_per_w, DIM), jnp.float32),
            pltpu.VMEM((b_per_w, LANES), jnp.float32),
            pltpu.VMEM((b_per_w,), jnp.float32),
            pltpu.SemaphoreType.DMA,
        ],
    )
    def sc_kernel(ent_hbm, rel_hbm, ih_hbm, ir_hbm, it_hbm, out_hbm,
                  ih_v, ir_v, it_v, h_v, r_v, t_v, acc_v, sum_v, sem):
        wid = lax.axis_index("s") * NCORES + lax.axis_index("c")
        row0 = wid * n_chunks

        pltpu.sync_copy(ih_hbm.at[pl.ds(row0, n_chunks)], ih_v)
        pltpu.sync_copy(ir_hbm.at[pl.ds(row0, n_chunks)], ir_v)
        pltpu.sync_copy(it_hbm.at[pl.ds(row0, n_chunks)], it_v)

        copies = []
        for j in range(n_chunks):
            dst = pl.ds(j * IDX_MINOR, IDX_MINOR)
            copies.append(pltpu.async_copy(ent_hbm.at[ih_v.at[j]], h_v.at[dst], sem))
            copies.append(pltpu.async_copy(rel_hbm.at[ir_v.at[j]], r_v.at[dst], sem))
            copies.append(pltpu.async_copy(ent_hbm.at[it_v.at[j]], t_v.at[dst], sem))
        for c in copies:
            c.wait()

        @pl.loop(0, b_per_w)
        def _(i):
            acc = None
            for c in range(DIM // LANES):
                sl = pl.ds(c * LANES, LANES)
                d = h_v[i, sl] + r_v[i, sl] - t_v[i, sl]
                acc = d * d if acc is None else acc + d * d
            acc_v[i, :] = acc

        lane_iota = lax.iota(jnp.int32, LANES)

        @pl.loop(0, b_per_w // LANES)
        def _(g):
            rows = g * LANES + lane_iota
            tot = None
            for c in range(LANES):
                cols = jnp.full((LANES,), c, jnp.int32)
                v = plsc.load_gather(acc_v, [rows, cols])
                tot = v if tot is None else tot + v
            sum_v[pl.ds(g * LANES, LANES)] = tot

        pltpu.sync_copy(sum_v, out_hbm.at[pl.ds(wid * b_per_w, b_per_w)])

    return sc_kernel(ent_emb, rel_emb, idx_h, idx_r, idx_t)


def _tc_score_body(p_ref, o_ref):
    o_ref[...] = -jnp.sqrt(p_ref[...] + 1e-12)


def _linearized(table):
    flat = lax.optimization_barrier(jnp.reshape(table, (-1,)))
    return jnp.reshape(flat, table.shape)


def kernel(batch_h, batch_r, batch_t, ent_emb, rel_emb):
    batch = batch_h.shape[0]
    shp = (batch // IDX_MINOR, IDX_MINOR)
    ih = batch_h.astype(jnp.int32).reshape(shp)
    ir = batch_r.astype(jnp.int32).reshape(shp)
    it = batch_t.astype(jnp.int32).reshape(shp)
    sumsq = _sc_sumsq(_linearized(ent_emb), _linearized(rel_emb),
                      ih, ir, it, batch)
    return pl.pallas_call(
        _tc_score_body,
        out_shape=jax.ShapeDtypeStruct((batch,), jnp.float32),
    )(sumsq)


# speedup vs baseline: 2.7973x; 2.7963x over previous
import functools

import jax
import jax.numpy as jnp
from jax import lax
from jax.experimental import pallas as pl
from jax.experimental.pallas import tpu as pltpu
from jax.experimental.pallas import tpu_sc as plsc

DIM = 64
LANES = 16
NCORES = 2
NSUBCORES = 16
DIMS_PER_CORE = DIM // NCORES
IDX_MINOR = 128


@functools.partial(jax.jit, static_argnames=("batch", "n_ent", "n_rel"))
def _sc_partial(ent_t, rel_t, idx_h, idx_r, idx_t, batch, n_ent, n_rel):
    b_per_t = batch // NSUBCORES
    n_rows = b_per_t // IDX_MINOR
    mesh = plsc.VectorSubcoreMesh(
        core_axis_name="c", subcore_axis_name="s",
        num_cores=NCORES, num_subcores=NSUBCORES,
    )

    idx_vmem = pltpu.VMEM((n_rows, IDX_MINOR), jnp.int32)

    @functools.partial(
        pl.kernel,
        out_type=jax.ShapeDtypeStruct((NCORES, batch), jnp.float32),
        mesh=mesh,
        compiler_params=pltpu.CompilerParams(needs_layout_passes=False),
        scratch_types=[
            pltpu.VMEM_SHARED((n_ent,), jnp.float32),
            pltpu.VMEM((DIMS_PER_CORE, n_rel), jnp.float32),
            idx_vmem, idx_vmem, idx_vmem,
            pltpu.VMEM((b_per_t,), jnp.float32),
            pltpu.VMEM((b_per_t,), jnp.float32),
            pltpu.VMEM((b_per_t,), jnp.float32),
            pltpu.SemaphoreType.DMA,
            pltpu.SemaphoreType.DMA,
        ],
    )
    def sc_kernel(ent_hbm, rel_hbm, ih_hbm, ir_hbm, it_hbm, out_hbm,
                  stage, rel_v, ih_v, ir_v, it_v,
                  hval_v, tval_v, acc_v, sem_s, sem_g):
        cid = lax.axis_index("c")
        sid = lax.axis_index("s")
        c_base = cid * DIMS_PER_CORE
        row0 = sid * n_rows

        pltpu.sync_copy(ih_hbm.at[pl.ds(row0, n_rows)], ih_v)
        pltpu.sync_copy(ir_hbm.at[pl.ds(row0, n_rows)], ir_v)
        pltpu.sync_copy(it_hbm.at[pl.ds(row0, n_rows)], it_v)
        pltpu.sync_copy(rel_hbm.at[pl.ds(c_base, DIMS_PER_CORE)], rel_v)

        zero = jnp.zeros((LANES,), jnp.float32)

        @pl.loop(0, b_per_t // LANES)
        def _(g):
            acc_v[pl.ds(g * LANES, LANES)] = zero

        @pl.when(sid == 0)
        def _():
            pltpu.async_copy(ent_hbm.at[c_base], stage, sem_s)

        @pl.loop(0, DIMS_PER_CORE)
        def _(c):
            @pl.when(sid == 0)
            def _():
                pltpu.make_async_copy(
                    ent_hbm.at[c_base + c], stage, sem_s).wait()

            plsc.subcore_barrier()

            gh = []
            for j in range(n_rows):
                dst = pl.ds(j * IDX_MINOR, IDX_MINOR)
                gh.append(pltpu.async_copy(
                    stage.at[ih_v.at[j]], hval_v.at[dst], sem_g))
                gh.append(pltpu.async_copy(
                    stage.at[it_v.at[j]], tval_v.at[dst], sem_g))
            for cp in gh:
                cp.wait()

            plsc.subcore_barrier()

            @pl.when((sid == 0) & (c < DIMS_PER_CORE - 1))
            def _():
                pltpu.async_copy(ent_hbm.at[c_base + c + 1], stage, sem_s)

            @pl.loop(0, n_rows)
            def _(j):
                for m in range(IDX_MINOR // LANES):
                    sl = pl.ds(j * IDX_MINOR + m * LANES, LANES)
                    isl = pl.ds(m * LANES, LANES)
                    rv = plsc.load_gather(
                        rel_v, [jax.lax.broadcast(c, (LANES,)),
                                ir_v[j, isl]])
                    d = hval_v[sl] + rv - tval_v[sl]
                    acc_v[sl] = acc_v[sl] + d * d

        pltpu.sync_copy(acc_v, out_hbm.at[cid, pl.ds(sid * b_per_t, b_per_t)])

    return sc_kernel(ent_t, rel_t, idx_h, idx_r, idx_t)


def _tc_score_body(p_ref, o_ref):
    o_ref[...] = -jnp.sqrt(p_ref[0, :] + p_ref[1, :] + 1e-12)


def kernel(batch_h, batch_r, batch_t, ent_emb, rel_emb):
    batch = batch_h.shape[0]
    shp = (batch // IDX_MINOR, IDX_MINOR)
    ih = batch_h.astype(jnp.int32).reshape(shp)
    ir = batch_r.astype(jnp.int32).reshape(shp)
    it = batch_t.astype(jnp.int32).reshape(shp)
    partial = _sc_partial(ent_emb.T, rel_emb.T, ih, ir, it,
                          batch, ent_emb.shape[0], rel_emb.shape[0])
    return pl.pallas_call(
        _tc_score_body,
        out_shape=jax.ShapeDtypeStruct((batch,), jnp.float32),
    )(partial)


# speedup vs baseline: 2.9963x; 1.0712x over previous
import functools

import jax
import jax.numpy as jnp
from jax import lax
from jax.experimental import pallas as pl
from jax.experimental.pallas import tpu as pltpu
from jax.experimental.pallas import tpu_sc as plsc

DIM = 64
LANES = 16
NCORES = 2
NSUBCORES = 16
DIMS_PER_CORE = DIM // NCORES
IDX_MINOR = 128


@functools.partial(jax.jit, static_argnames=("batch", "n_ent", "n_rel", "lo_n"))
def _sc_partial(ent_t, rel_t, tail_t, ih_lo, ih_hi, ih_ta, it_lo, it_hi,
                it_ta, idx_r, batch, n_ent, n_rel, lo_n):
    b_per_t = batch // NSUBCORES
    n_rows = b_per_t // IDX_MINOR
    stage_n = lo_n + IDX_MINOR
    mesh = plsc.VectorSubcoreMesh(
        core_axis_name="c", subcore_axis_name="s",
        num_cores=NCORES, num_subcores=NSUBCORES,
    )

    idx_vmem = pltpu.VMEM((n_rows, IDX_MINOR), jnp.int32)
    val_vmem = pltpu.VMEM((b_per_t,), jnp.float32)

    @functools.partial(
        pl.kernel,
        out_type=jax.ShapeDtypeStruct((NCORES, batch), jnp.float32),
        mesh=mesh,
        compiler_params=pltpu.CompilerParams(needs_layout_passes=False),
        scratch_types=[
            pltpu.VMEM_SHARED((stage_n,), jnp.float32),
            pltpu.VMEM_SHARED((stage_n,), jnp.float32),
            pltpu.VMEM((DIMS_PER_CORE, n_rel), jnp.float32),
            pltpu.VMEM((DIMS_PER_CORE, IDX_MINOR), jnp.float32),
            idx_vmem, idx_vmem, idx_vmem,
            idx_vmem, idx_vmem, idx_vmem,
            idx_vmem,
            val_vmem, val_vmem, val_vmem, val_vmem,
            val_vmem,
            pltpu.SemaphoreType.DMA,
            pltpu.SemaphoreType.DMA,
            pltpu.SemaphoreType.DMA,
        ],
    )
    def sc_kernel(ent_hbm, rel_hbm, tail_hbm,
                  ihl_hbm, ihh_hbm, iht_hbm, itl_hbm, ith_hbm, itt_hbm,
                  ir_hbm, out_hbm,
                  stage_a, stage_b, rel_v, tail_v,
                  ihl_v, ihh_v, iht_v, itl_v, ith_v, itt_v, ir_v,
                  hlo_v, hhi_v, tlo_v, thi_v, acc_v,
                  sem_a, sem_b, sem_g):
        cid = lax.axis_index("c")
        sid = lax.axis_index("s")
        c_base = cid * DIMS_PER_CORE
        row0 = sid * n_rows

        pltpu.sync_copy(ihl_hbm.at[pl.ds(row0, n_rows)], ihl_v)
        pltpu.sync_copy(ihh_hbm.at[pl.ds(row0, n_rows)], ihh_v)
        pltpu.sync_copy(iht_hbm.at[pl.ds(row0, n_rows)], iht_v)
        pltpu.sync_copy(itl_hbm.at[pl.ds(row0, n_rows)], itl_v)
        pltpu.sync_copy(ith_hbm.at[pl.ds(row0, n_rows)], ith_v)
        pltpu.sync_copy(itt_hbm.at[pl.ds(row0, n_rows)], itt_v)
        pltpu.sync_copy(ir_hbm.at[pl.ds(row0, n_rows)], ir_v)
        pltpu.sync_copy(rel_hbm.at[pl.ds(c_base, DIMS_PER_CORE)], rel_v)
        pltpu.sync_copy(tail_hbm.at[pl.ds(c_base, DIMS_PER_CORE)], tail_v)

        zero = jnp.zeros((LANES,), jnp.float32)

        @pl.loop(0, b_per_t // LANES)
        def _(g):
            acc_v[pl.ds(g * LANES, LANES)] = zero

        @pl.when(sid == 0)
        def _():
            pltpu.sync_copy(acc_v.at[pl.ds(0, IDX_MINOR)],
                            stage_a.at[pl.ds(lo_n, IDX_MINOR)])
            pltpu.async_copy(
                ent_hbm.at[c_base].at[pl.ds(0, lo_n)],
                stage_a.at[pl.ds(0, lo_n)], sem_a)

        @pl.when(sid == 1)
        def _():
            pltpu.sync_copy(acc_v.at[pl.ds(0, IDX_MINOR)],
                            stage_b.at[pl.ds(lo_n, IDX_MINOR)])
            pltpu.async_copy(
                ent_hbm.at[c_base].at[pl.ds(lo_n, lo_n)],
                stage_b.at[pl.ds(0, lo_n)], sem_b)

        def gather_vals(stage, iv_h, iv_t, dst_h, dst_t):
            gh = []
            for j in range(n_rows):
                dst = pl.ds(j * IDX_MINOR, IDX_MINOR)
                gh.append(pltpu.async_copy(
                    stage.at[iv_h.at[j]], dst_h.at[dst], sem_g))
                gh.append(pltpu.async_copy(
                    stage.at[iv_t.at[j]], dst_t.at[dst], sem_g))
            for cp in gh:
                cp.wait()

        @pl.loop(0, DIMS_PER_CORE)
        def _(c):
            @pl.when(sid == 0)
            def _():
                pltpu.make_async_copy(
                    ent_hbm.at[c_base + c].at[pl.ds(0, lo_n)],
                    stage_a.at[pl.ds(0, lo_n)], sem_a).wait()

            plsc.subcore_barrier()

            gather_vals(stage_a, ihl_v, itl_v, hlo_v, tlo_v)

            @pl.when(sid == 1)
            def _():
                pltpu.make_async_copy(
                    ent_hbm.at[c_base + c].at[pl.ds(lo_n, lo_n)],
                    stage_b.at[pl.ds(0, lo_n)], sem_b).wait()

            plsc.subcore_barrier()

            @pl.when((sid == 0) & (c < DIMS_PER_CORE - 1))
            def _():
                pltpu.async_copy(
                    ent_hbm.at[c_base + c + 1].at[pl.ds(0, lo_n)],
                    stage_a.at[pl.ds(0, lo_n)], sem_a)

            gather_vals(stage_b, ihh_v, ith_v, hhi_v, thi_v)

            plsc.subcore_barrier()

            @pl.when((sid == 1) & (c < DIMS_PER_CORE - 1))
            def _():
                pltpu.async_copy(
                    ent_hbm.at[c_base + c + 1].at[pl.ds(lo_n, lo_n)],
                    stage_b.at[pl.ds(0, lo_n)], sem_b)

            @pl.loop(0, n_rows)
            def _(j):
                for m in range(IDX_MINOR // LANES):
                    sl = pl.ds(j * IDX_MINOR + m * LANES, LANES)
                    isl = pl.ds(m * LANES, LANES)
                    cvec = jax.lax.broadcast(c, (LANES,))
                    rv = plsc.load_gather(rel_v, [cvec, ir_v[j, isl]])
                    htv = plsc.load_gather(tail_v, [cvec, iht_v[j, isl]])
                    ttv = plsc.load_gather(tail_v, [cvec, itt_v[j, isl]])
                    d = (hlo_v[sl] + hhi_v[sl] + htv + rv
                         - tlo_v[sl] - thi_v[sl] - ttv)
                    acc_v[sl] = acc_v[sl] + d * d

        pltpu.sync_copy(acc_v, out_hbm.at[cid, pl.ds(sid * b_per_t, b_per_t)])

    return sc_kernel(ent_t, rel_t, tail_t, ih_lo, ih_hi, ih_ta,
                     it_lo, it_hi, it_ta, idx_r)


def _tc_score_body(p_ref, o_ref):
    o_ref[...] = -jnp.sqrt(p_ref[0, :] + p_ref[1, :] + 1e-12)


def kernel(batch_h, batch_r, batch_t, ent_emb, rel_emb):
    batch = batch_h.shape[0]
    n_ent = ent_emb.shape[0]
    lo_n = (n_ent // (2 * IDX_MINOR)) * IDX_MINOR
    mid = 2 * lo_n
    tail_n = n_ent - mid
    shp = (batch // IDX_MINOR, IDX_MINOR)
    pos = jnp.arange(batch, dtype=jnp.int32)

    def split_idx(idx):
        idx = idx.astype(jnp.int32)
        pad = lo_n + (pos % IDX_MINOR)
        lo = jnp.where(idx < lo_n, idx, pad)
        hi = jnp.where((idx >= lo_n) & (idx < mid), idx - lo_n, pad)
        ta = jnp.where(idx >= mid, idx - mid,
                       tail_n + (pos % (IDX_MINOR - tail_n)))
        return lo.reshape(shp), hi.reshape(shp), ta.reshape(shp)

    ih_lo, ih_hi, ih_ta = split_idx(batch_h)
    it_lo, it_hi, it_ta = split_idx(batch_t)
    ir = batch_r.astype(jnp.int32).reshape(shp)
    tail_t = jnp.pad(ent_emb[mid:].T, ((0, 0), (0, IDX_MINOR - tail_n)))
    partial = _sc_partial(ent_emb.T, rel_emb.T, tail_t,
                          ih_lo, ih_hi, ih_ta, it_lo, it_hi, it_ta, ir,
                          batch, n_ent, rel_emb.shape[0], lo_n)
    return pl.pallas_call(
        _tc_score_body,
        out_shape=jax.ShapeDtypeStruct((batch,), jnp.float32),
    )(partial)
